# baseline (speedup 1.0000x reference)
import functools

import jax
import jax.numpy as jnp
from jax import lax
from jax.experimental import pallas as pl
from jax.experimental.pallas import tpu as pltpu
from jax.experimental.pallas import tpu_sc as plsc

D = 64
L = 16
NC, NS = 2, 16
NW = NC * NS
IDXW = 128
CHUNK = 256
EPS = 1e-5


def _rsqrt(x):
    i = plsc.bitcast(x, jnp.int32)
    i = jnp.int32(0x5F3759DF) - lax.shift_right_logical(i, 1)
    y = plsc.bitcast(i, jnp.float32)
    h = x * jnp.float32(-0.5)
    for _ in range(3):
        y = y * (jnp.float32(1.5) + h * y * y)
    return y


def _bcast(s):
    return lax.broadcast_in_dim(s, (L,), ())


def _make_sc_kernel(total_rows):
    per_w = total_rows // NW
    nchunks = per_w // CHUNK
    slabs_per_w = per_w // IDXW
    slabs_per_chunk = CHUNK // IDXW
    assert per_w * NW == total_rows and nchunks * CHUNK == per_w

    mesh = plsc.VectorSubcoreMesh(core_axis_name="c", subcore_axis_name="s")

    @functools.partial(
        pl.kernel,
        out_type=jax.ShapeDtypeStruct((total_rows, D), jnp.float32),
        mesh=mesh,
        compiler_params=pltpu.CompilerParams(
            needs_layout_passes=False, use_tc_tiling_on_sc=False),
        scratch_types=[
            pltpu.VMEM((slabs_per_w, IDXW), jnp.int32),
            pltpu.VMEM((CHUNK, D), jnp.float32),
            pltpu.VMEM((CHUNK, D), jnp.float32),
            pltpu.VMEM((D,), jnp.float32),
            pltpu.VMEM((D,), jnp.float32),
            pltpu.SemaphoreType.DMA,
        ],
    )
    def sc_kernel(x_hbm, table_hbm, gamma_hbm, beta_hbm, out_hbm,
                  idx_v, rows_v, outb_v, gam_v, bet_v, gsem):
        wid = lax.axis_index("s") * NC + lax.axis_index("c")
        pltpu.sync_copy(gamma_hbm, gam_v)
        pltpu.sync_copy(beta_hbm, bet_v)
        pltpu.sync_copy(x_hbm.at[pl.ds(wid * slabs_per_w, slabs_per_w)], idx_v)

        gam = [gam_v[pl.ds(j * L, L)] for j in range(D // L)]
        bet = [bet_v[pl.ds(j * L, L)] for j in range(D // L)]

        def ln_row(r, carry):
            v = [rows_v[r, pl.ds(j * L, L)] for j in range(D // L)]
            vs = v[0] + v[1] + v[2] + v[3]
            vq = v[0] * v[0] + v[1] * v[1] + v[2] * v[2] + v[3] * v[3]
            sv = _bcast(jnp.sum(vs))
            qv = _bcast(jnp.sum(vq))
            meanv = sv * jnp.float32(1.0 / D)
            varv = qv * jnp.float32(1.0 / D) - meanv * meanv
            rstd = _rsqrt(jnp.maximum(varv, jnp.float32(0.0)) + jnp.float32(EPS))
            for j in range(D // L):
                outb_v[r, pl.ds(j * L, L)] = (v[j] - meanv) * (rstd * gam[j]) + bet[j]
            return carry

        def chunk_body(g, carry):
            waits = []
            for s in range(slabs_per_chunk):
                waits.append(pltpu.async_copy(
                    table_hbm.at[idx_v.at[g * slabs_per_chunk + s]],
                    rows_v.at[pl.ds(s * IDXW, IDXW)],
                    gsem))
            for w in waits:
                w.wait()
            lax.fori_loop(0, CHUNK, ln_row, 0, unroll=2)
            pltpu.sync_copy(
                outb_v,
                out_hbm.at[pl.ds(wid * per_w + g * CHUNK, CHUNK)])
            return carry

        lax.fori_loop(0, nchunks, chunk_body, 0)

    return sc_kernel


def kernel(x, table, gamma, beta):
    b, h = x.shape
    total = b * h
    x2d = x.reshape(total // IDXW, IDXW).astype(jnp.int32)
    out = _make_sc_kernel(total)(x2d, table, gamma, beta)
    return out.reshape(b, h, D)


# speedup vs baseline: 1.7721x; 1.7721x over previous
import functools

import jax
import jax.numpy as jnp
from jax import lax
from jax.experimental import pallas as pl
from jax.experimental.pallas import tpu as pltpu
from jax.experimental.pallas import tpu_sc as plsc

D = 64
L = 16
NC, NS = 2, 16
NW = NC * NS
IDXW = 128
CHUNK = 256
EPS = 1e-5


def _rsqrt(x):
    i = plsc.bitcast(x, jnp.int32)
    i = jnp.int32(0x5F3759DF) - lax.shift_right_logical(i, 1)
    y = plsc.bitcast(i, jnp.float32)
    h = x * jnp.float32(-0.5)
    for _ in range(2):
        y = y * (jnp.float32(1.5) + h * y * y)
    return y


def _bcast(s):
    return lax.broadcast_in_dim(s, (L,), ())


def _make_sc_kernel(total_rows):
    per_w = total_rows // NW
    nchunks = per_w // CHUNK
    slabs_per_w = per_w // IDXW
    spc = CHUNK // IDXW
    assert per_w * NW == total_rows and nchunks * CHUNK == per_w
    assert nchunks % 2 == 0 and nchunks >= 4

    mesh = plsc.VectorSubcoreMesh(core_axis_name="c", subcore_axis_name="s")

    @functools.partial(
        pl.kernel,
        out_type=jax.ShapeDtypeStruct((total_rows, D), jnp.float32),
        mesh=mesh,
        compiler_params=pltpu.CompilerParams(
            needs_layout_passes=False, use_tc_tiling_on_sc=False),
        scratch_types=[
            pltpu.VMEM((slabs_per_w, IDXW), jnp.int32),
            pltpu.VMEM((CHUNK, D), jnp.float32),
            pltpu.VMEM((CHUNK, D), jnp.float32),
            pltpu.VMEM((CHUNK, D), jnp.float32),
            pltpu.VMEM((CHUNK, D), jnp.float32),
            pltpu.VMEM((D,), jnp.float32),
            pltpu.VMEM((D,), jnp.float32),
            pltpu.SemaphoreType.DMA,
            pltpu.SemaphoreType.DMA,
            pltpu.SemaphoreType.DMA,
            pltpu.SemaphoreType.DMA,
        ],
    )
    def sc_kernel(x_hbm, table_hbm, gamma_hbm, beta_hbm, out_hbm,
                  idx_v, rows0, rows1, outb0, outb1, gam_v, bet_v,
                  gsem0, gsem1, osem0, osem1):
        wid = lax.axis_index("s") * NC + lax.axis_index("c")
        rows = (rows0, rows1)
        outb = (outb0, outb1)
        gsem = (gsem0, gsem1)
        osem = (osem0, osem1)

        pltpu.sync_copy(gamma_hbm, gam_v)
        pltpu.sync_copy(beta_hbm, bet_v)
        pltpu.sync_copy(x_hbm.at[pl.ds(wid * slabs_per_w, slabs_per_w)], idx_v)

        gam = [gam_v[pl.ds(j * L, L)] for j in range(D // L)]
        bet = [bet_v[pl.ds(j * L, L)] for j in range(D // L)]

        def start_gather(g, b):
            for s in range(spc):
                pltpu.async_copy(
                    table_hbm.at[idx_v.at[g * spc + s]],
                    rows[b].at[pl.ds(s * IDXW, IDXW)],
                    gsem[b])

        def wait_gather(b):
            pltpu.make_async_copy(
                table_hbm.at[pl.ds(0, CHUNK)], rows[b], gsem[b]).wait()

        def start_scatter(g, b):
            pltpu.async_copy(
                outb[b],
                out_hbm.at[pl.ds(wid * per_w + g * CHUNK, CHUNK)],
                osem[b])

        def wait_scatter(b):
            pltpu.make_async_copy(
                outb[b], out_hbm.at[pl.ds(0, CHUNK)], osem[b]).wait()

        def compute(b):
            rv, ov = rows[b], outb[b]

            @plsc.parallel_loop(0, CHUNK, unroll=4)
            def ln_row(r):
                v = [rv[r, pl.ds(j * L, L)] for j in range(D // L)]
                vs = (v[0] + v[1]) + (v[2] + v[3])
                vq = (v[0] * v[0] + v[1] * v[1]) + (v[2] * v[2] + v[3] * v[3])
                sv = _bcast(jnp.sum(vs))
                qv = _bcast(jnp.sum(vq))
                meanv = sv * jnp.float32(1.0 / D)
                varv = qv * jnp.float32(1.0 / D) - meanv * meanv
                rstd = _rsqrt(jnp.maximum(varv, jnp.float32(0.0))
                              + jnp.float32(EPS))
                for j in range(D // L):
                    ov[r, pl.ds(j * L, L)] = \
                        (v[j] - meanv) * (rstd * gam[j]) + bet[j]

        start_gather(0, 0)
        start_gather(1, 1)
        for g in (0, 1):
            wait_gather(g)
            compute(g)
            start_scatter(g, g)
            start_gather(g + 2, g)

        def pair_body(i, carry):
            for b in range(2):
                g = 2 * i + b
                wait_gather(b)
                wait_scatter(b)
                compute(b)
                start_scatter(g, b)
                start_gather(g + 2, b)
            return carry

        lax.fori_loop(1, nchunks // 2 - 1, pair_body, 0)

        for b in range(2):
            g = nchunks - 2 + b
            wait_gather(b)
            wait_scatter(b)
            compute(b)
            start_scatter(g, b)
        for b in range(2):
            wait_scatter(b)

    return sc_kernel


def kernel(x, table, gamma, beta):
    b, h = x.shape
    total = b * h
    x2d = x.reshape(total // IDXW, IDXW).astype(jnp.int32)
    out = _make_sc_kernel(total)(x2d, table, gamma, beta)
    return out.reshape(b, h, D)
